# baseline (speedup 1.0000x reference)
import functools

import jax
import jax.numpy as jnp
from jax import lax
from jax.experimental import pallas as pl
from jax.experimental.pallas import tpu as pltpu
from jax.experimental.pallas import tpu_sc as plsc

VOCAB = 1000000
EMBED_DIM = 32
BATCH = 16384

NUM_CORES = 2
NUM_SUBCORES = 16
B_PER_S = BATCH // NUM_SUBCORES
DIMS_PER_CORE = EMBED_DIM // NUM_CORES
HALF = B_PER_S // 2
VMAIN = 983040
VTAIL = VOCAB - VMAIN
CHUNK = 128
N_CHUNKS = HALF // CHUNK
LANES = 16
N_VREG = HALF // LANES


def _glove_dot_sc(table3d, target_idx, context_idx):
  mesh = plsc.VectorSubcoreMesh(core_axis_name="c", subcore_axis_name="s")

  @functools.partial(
      pl.kernel,
      mesh=mesh,
      compiler_params=pltpu.CompilerParams(needs_layout_passes=False),
      out_type=jax.ShapeDtypeStruct((NUM_CORES, BATCH), jnp.float32),
      scratch_types=[
          pltpu.VMEM_SHARED((VMAIN,), jnp.float32),
          pltpu.VMEM_SHARED((VMAIN,), jnp.float32),
          pltpu.VMEM_SHARED((VTAIL,), jnp.float32),
          pltpu.VMEM_SHARED((VTAIL,), jnp.float32),
          pltpu.VMEM((2 * N_CHUNKS, CHUNK), jnp.int32),
          pltpu.VMEM((2 * N_CHUNKS, CHUNK), jnp.int32),
          pltpu.VMEM((N_CHUNKS, CHUNK), jnp.int32),
          pltpu.VMEM((N_CHUNKS, CHUNK), jnp.int32),
          pltpu.VMEM((HALF,), jnp.float32),
          pltpu.VMEM((HALF,), jnp.float32),
          pltpu.VMEM((B_PER_S,), jnp.float32),
          pltpu.SemaphoreType.DMA,
          pltpu.SemaphoreType.DMA,
      ],
  )
  def k(t3_hbm, tidx_hbm, cidx_hbm, out_hbm,
        main0, main1, tail0, tail1, tidx_v, cidx_v, tcl_v, ccl_v,
        tval_v, cval_v, acc_v, sem_s, sem_g):
    mains = (main0, main1)
    tails = (tail0, tail1)
    core = lax.axis_index("c")
    sub = lax.axis_index("s")

    def zero_acc(v8, carry):
      acc_v[pl.ds(v8 * LANES, LANES)] = jnp.zeros((LANES,), jnp.float32)
      return carry
    lax.fori_loop(0, B_PER_S // LANES, zero_acc, 0, unroll=8)

    pltpu.sync_copy(tidx_hbm.at[pl.ds(sub * 2 * N_CHUNKS, 2 * N_CHUNKS)],
                    tidx_v)
    pltpu.sync_copy(cidx_hbm.at[pl.ds(sub * 2 * N_CHUNKS, 2 * N_CHUNKS)],
                    cidx_v)

    VSLICE = VMAIN // NUM_SUBCORES

    NSUB = 4
    SUBSL = VSLICE // NSUB

    def stream_copies(d, parity):
      gg = 2 * core + lax.shift_right_logical(d, 3)
      ss = d & 7
      cps = [
          pltpu.make_async_copy(
              t3_hbm.at[gg, ss, pl.ds(sub * VSLICE + j * SUBSL, SUBSL)],
              mains[parity].at[pl.ds(sub * VSLICE + j * SUBSL, SUBSL)],
              sem_s)
          for j in range(NSUB)
      ]
      cps.append(pltpu.make_async_copy(
          t3_hbm.at[gg, ss, pl.ds(VMAIN, VTAIL)], tails[parity], sem_s))
      return cps

    def fire(d, parity):
      cps = stream_copies(d, parity)

      @pl.when(sub == 0)
      def _():
        cps[-1].start()
      for cp in cps[:-1]:
        cp.start()

    def drain(d, parity):
      cps = stream_copies(d, parity)
      for cp in cps[:-1]:
        cp.wait()

      @pl.when(sub == 0)
      def _():
        cps[-1].wait()

    def process_dim(d, parity):
      main = mains[parity]
      tail = tails[parity]
      drain(d, parity)
      plsc.subcore_barrier()
      for h in range(2):
        def clamp_main(v8, carry):
          ch = lax.shift_right_logical(v8, 3)
          sl = pl.ds((v8 & 7) * LANES, LANES)
          tcl_v[ch, sl] = jnp.minimum(tidx_v[h * N_CHUNKS + ch, sl],
                                      VMAIN - 1)
          ccl_v[ch, sl] = jnp.minimum(cidx_v[h * N_CHUNKS + ch, sl],
                                      VMAIN - 1)
          return carry
        lax.fori_loop(0, HALF // LANES, clamp_main, 0, unroll=8)
        gathers = []
        for ch in range(N_CHUNKS):
          gathers.append(pltpu.async_copy(
              main.at[tcl_v.at[ch]],
              tval_v.at[pl.ds(ch * CHUNK, CHUNK)], sem_g))
          gathers.append(pltpu.async_copy(
              main.at[ccl_v.at[ch]],
              cval_v.at[pl.ds(ch * CHUNK, CHUNK)], sem_g))
        for cp in gathers:
          cp.wait()

        def clamp_tail(v8, carry):
          ch = lax.shift_right_logical(v8, 3)
          sl = pl.ds((v8 & 7) * LANES, LANES)
          ti = tidx_v[h * N_CHUNKS + ch, sl]
          ci = cidx_v[h * N_CHUNKS + ch, sl]
          tcl_v[ch, sl] = jnp.where(ti >= VMAIN, ti - VMAIN, -1)
          ccl_v[ch, sl] = jnp.where(ci >= VMAIN, ci - VMAIN, -1)
          return carry
        lax.fori_loop(0, HALF // LANES, clamp_tail, 0, unroll=8)
        tgat = []
        for ch in range(N_CHUNKS):
          tgat.append(pltpu.async_copy(
              tail.at[plsc.Indices(tcl_v.at[ch], ignored_value=-1)],
              tval_v.at[pl.ds(ch * CHUNK, CHUNK)], sem_g))
          tgat.append(pltpu.async_copy(
              tail.at[plsc.Indices(ccl_v.at[ch], ignored_value=-1)],
              cval_v.at[pl.ds(ch * CHUNK, CHUNK)], sem_g))
        for cp in tgat:
          cp.wait()

        def accum(v8, carry):
          sl = pl.ds(v8 * LANES, LANES)
          asl = pl.ds(h * HALF + v8 * LANES, LANES)
          acc_v[asl] = acc_v[asl] + tval_v[sl] * cval_v[sl]
          return carry
        lax.fori_loop(0, N_VREG, accum, 0, unroll=8)

      plsc.subcore_barrier()

    n_pairs = DIMS_PER_CORE // 2
    fire(jnp.int32(0), 0)
    fire(jnp.int32(1), 1)

    def pair_body(p, carry):
      process_dim(2 * p, 0)

      @pl.when(p + 1 < n_pairs)
      def _():
        fire(2 * p + 2, 0)

      process_dim(2 * p + 1, 1)

      @pl.when(p + 1 < n_pairs)
      def _():
        fire(2 * p + 3, 1)

      return carry

    lax.fori_loop(0, n_pairs, pair_body, 0)

    pltpu.sync_copy(acc_v, out_hbm.at[core, pl.ds(sub * B_PER_S, B_PER_S)])

  tidx2d = target_idx.reshape(BATCH // CHUNK, CHUNK)
  cidx2d = context_idx.reshape(BATCH // CHUNK, CHUNK)
  return k(table3d, tidx2d, cidx2d)


def kernel(target, context, table):
  t = target.astype(jnp.int32)
  c = context.astype(jnp.int32)
  table3d = table.T.reshape(4, 8, VOCAB)
  parts = _glove_dot_sc(table3d, t, c)
  return (parts[0] + parts[1]).reshape(BATCH, 1)


# speedup vs baseline: 1.0314x; 1.0314x over previous
import functools

import jax
import jax.numpy as jnp
from jax import lax
from jax.experimental import pallas as pl
from jax.experimental.pallas import tpu as pltpu
from jax.experimental.pallas import tpu_sc as plsc

VOCAB = 1000000
EMBED_DIM = 32
BATCH = 16384

NUM_CORES = 2
NUM_SUBCORES = 16
B_PER_S = BATCH // NUM_SUBCORES
DIMS_PER_CORE = EMBED_DIM // NUM_CORES
HALF = B_PER_S // 2
VMAIN = 983040
VTAIL = VOCAB - VMAIN
CHUNK = 128
N_CHUNKS = HALF // CHUNK
LANES = 16
N_VREG = HALF // LANES


def _glove_dot_sc(table3d, target_idx, context_idx):
  mesh = plsc.VectorSubcoreMesh(core_axis_name="c", subcore_axis_name="s")

  @functools.partial(
      pl.kernel,
      mesh=mesh,
      compiler_params=pltpu.CompilerParams(needs_layout_passes=False),
      out_type=jax.ShapeDtypeStruct((NUM_CORES, BATCH), jnp.float32),
      scratch_types=[
          pltpu.VMEM_SHARED((VMAIN,), jnp.float32),
          pltpu.VMEM_SHARED((VMAIN,), jnp.float32),
          pltpu.VMEM_SHARED((VTAIL,), jnp.float32),
          pltpu.VMEM_SHARED((VTAIL,), jnp.float32),
          pltpu.VMEM((2 * N_CHUNKS, CHUNK), jnp.int32),
          pltpu.VMEM((2 * N_CHUNKS, CHUNK), jnp.int32),
          pltpu.VMEM((N_CHUNKS, CHUNK), jnp.int32),
          pltpu.VMEM((N_CHUNKS, CHUNK), jnp.int32),
          pltpu.VMEM((HALF,), jnp.float32),
          pltpu.VMEM((HALF,), jnp.float32),
          pltpu.VMEM((B_PER_S,), jnp.float32),
          pltpu.SemaphoreType.DMA,
          pltpu.SemaphoreType.DMA,
      ],
  )
  def k(t3_hbm, tidx_hbm, cidx_hbm, out_hbm,
        main0, main1, tail0, tail1, tidx_v, cidx_v, tcl_v, ccl_v,
        tval_v, cval_v, acc_v, sem_s, sem_g):
    mains = (main0, main1)
    tails = (tail0, tail1)
    core = lax.axis_index("c")
    sub = lax.axis_index("s")

    def zero_acc(v8, carry):
      acc_v[pl.ds(v8 * LANES, LANES)] = jnp.zeros((LANES,), jnp.float32)
      return carry
    lax.fori_loop(0, B_PER_S // LANES, zero_acc, 0)

    pltpu.sync_copy(tidx_hbm.at[pl.ds(sub * 2 * N_CHUNKS, 2 * N_CHUNKS)],
                    tidx_v)
    pltpu.sync_copy(cidx_hbm.at[pl.ds(sub * 2 * N_CHUNKS, 2 * N_CHUNKS)],
                    cidx_v)

    VSLICE = VMAIN // NUM_SUBCORES

    def stream_copies(d, parity):
      gg = 2 * core + lax.shift_right_logical(d, 3)
      ss = d & 7
      return (
          pltpu.make_async_copy(
              t3_hbm.at[gg, ss, pl.ds(sub * VSLICE + 0, VSLICE)],
              mains[parity].at[pl.ds(sub * VSLICE + 0, VSLICE)], sem_s),
          pltpu.make_async_copy(
              t3_hbm.at[gg, ss, pl.ds(VMAIN, VTAIL)], tails[parity], sem_s),
      )

    def fire(d, parity):
      cps = stream_copies(d, parity)
      cps[0].start()

      @pl.when(sub == 0)
      def _():
        cps[1].start()

    def drain(d, parity):
      cps = stream_copies(d, parity)
      cps[0].wait()

      @pl.when(sub == 0)
      def _():
        cps[1].wait()

    def process_dim(d, parity):
      main = mains[parity]
      tail = tails[parity]
      drain(d, parity)
      plsc.subcore_barrier()
      for h in range(2):
        def clamp_main(v8, carry):
          ch = lax.shift_right_logical(v8, 3)
          sl = pl.ds((v8 & 7) * LANES, LANES)
          tcl_v[ch, sl] = jnp.minimum(tidx_v[h * N_CHUNKS + ch, sl],
                                      VMAIN - 1)
          ccl_v[ch, sl] = jnp.minimum(cidx_v[h * N_CHUNKS + ch, sl],
                                      VMAIN - 1)
          return carry
        lax.fori_loop(0, HALF // LANES, clamp_main, 0)
        gathers = []
        for ch in range(N_CHUNKS):
          gathers.append(pltpu.async_copy(
              main.at[tcl_v.at[ch]],
              tval_v.at[pl.ds(ch * CHUNK, CHUNK)], sem_g))
          gathers.append(pltpu.async_copy(
              main.at[ccl_v.at[ch]],
              cval_v.at[pl.ds(ch * CHUNK, CHUNK)], sem_g))
        for cp in gathers:
          cp.wait()

        def clamp_tail(v8, carry):
          ch = lax.shift_right_logical(v8, 3)
          sl = pl.ds((v8 & 7) * LANES, LANES)
          ti = tidx_v[h * N_CHUNKS + ch, sl]
          ci = cidx_v[h * N_CHUNKS + ch, sl]
          tcl_v[ch, sl] = jnp.where(ti >= VMAIN, ti - VMAIN, -1)
          ccl_v[ch, sl] = jnp.where(ci >= VMAIN, ci - VMAIN, -1)
          return carry
        lax.fori_loop(0, HALF // LANES, clamp_tail, 0)
        tgat = []
        for ch in range(N_CHUNKS):
          tgat.append(pltpu.async_copy(
              tail.at[plsc.Indices(tcl_v.at[ch], ignored_value=-1)],
              tval_v.at[pl.ds(ch * CHUNK, CHUNK)], sem_g))
          tgat.append(pltpu.async_copy(
              tail.at[plsc.Indices(ccl_v.at[ch], ignored_value=-1)],
              cval_v.at[pl.ds(ch * CHUNK, CHUNK)], sem_g))
        for cp in tgat:
          cp.wait()

        def accum(v8, carry):
          sl = pl.ds(v8 * LANES, LANES)
          asl = pl.ds(h * HALF + v8 * LANES, LANES)
          acc_v[asl] = acc_v[asl] + tval_v[sl] * cval_v[sl]
          return carry
        lax.fori_loop(0, N_VREG, accum, 0)

      plsc.subcore_barrier()

    n_pairs = DIMS_PER_CORE // 2
    fire(jnp.int32(0), 0)
    fire(jnp.int32(1), 1)

    def pair_body(p, carry):
      process_dim(2 * p, 0)

      @pl.when(p + 1 < n_pairs)
      def _():
        fire(2 * p + 2, 0)

      process_dim(2 * p + 1, 1)

      @pl.when(p + 1 < n_pairs)
      def _():
        fire(2 * p + 3, 1)

      return carry

    lax.fori_loop(0, n_pairs, pair_body, 0)

    pltpu.sync_copy(acc_v, out_hbm.at[core, pl.ds(sub * B_PER_S, B_PER_S)])

  tidx2d = target_idx.reshape(BATCH // CHUNK, CHUNK)
  cidx2d = context_idx.reshape(BATCH // CHUNK, CHUNK)
  return k(table3d, tidx2d, cidx2d)


def kernel(target, context, table):
  t = target.astype(jnp.int32)
  c = context.astype(jnp.int32)
  table3d = table.T.reshape(4, 8, VOCAB)
  parts = _glove_dot_sc(table3d, t, c)
  return (parts[0] + parts[1]).reshape(BATCH, 1)
